# baseline (speedup 1.0000x reference)
import functools

import jax
import jax.numpy as jnp
from jax import lax
from jax.experimental import pallas as pl
from jax.experimental.pallas import tpu as pltpu
from jax.experimental.pallas import tpu_sc as plsc

N = 10000
E = 320000
D_IN = 128
NHID = 128
NCLASS = 16
DIM2 = NCLASS + 2
D2P = 32

NTILES = 16
CHUNK = 64
RB = 4
CPT = 320
IB = 20
NIB = CPT // IB
IRING = 2 * IB
EPT = CPT * CHUNK
EPAD = EPT * NTILES
NACC = 10016
ZPT = NACC // NTILES
OPT = N // NTILES
BCH = CHUNK


def _mm_body(x0_ref, x1_ref, w0_ref, w1_ref, o0_ref, o1_ref):
    o0_ref[...] = jnp.dot(x0_ref[...], w0_ref[...], preferred_element_type=jnp.float32)
    o1_ref[...] = jnp.dot(x1_ref[...], w1_ref[...], preferred_element_type=jnp.float32)


def _layer2_body(y0_ref, y1_ref, b1_ref, w_ref, b2_ref, o0_ref, o1_ref):
    for y_ref, o_ref in ((y0_ref, o0_ref), (y1_ref, o1_ref)):
        h = jnp.maximum(y_ref[...] + 2.0 * b1_ref[...], 0.0)
        o_ref[...] = jnp.dot(h, w_ref[...], preferred_element_type=jnp.float32) + b2_ref[...]


def _logsoftmax_body(z0_ref, z1_ref, o0_ref, o1_ref):
    for z_ref, o_ref in ((z0_ref, o0_ref), (z1_ref, o1_ref)):
        z = z_ref[...]
        col = lax.broadcasted_iota(jnp.int32, z.shape, 1)
        valid = col < DIM2
        zm = jnp.where(valid, z, -jnp.inf)
        m = jnp.max(zm, axis=1, keepdims=True)
        e = jnp.where(valid, jnp.exp(z - m), 0.0)
        s = jnp.sum(e, axis=1, keepdims=True)
        o_ref[...] = ((z - m) - jnp.log(s))[:, :DIM2]


def _chunks(total, step):
    out = []
    while total > 0:
        out.append(min(step, total))
        total -= out[-1]
    return tuple(out)


def _make_spmm(d, staged=False):
    mesh = plsc.VectorSubcoreMesh(core_axis_name="c", subcore_axis_name="s")

    @functools.partial(
        pl.kernel,
        out_type=[jax.ShapeDtypeStruct((N, d), jnp.float32),
                  jax.ShapeDtypeStruct((N, d), jnp.float32)],
        mesh=mesh,
        compiler_params=pltpu.CompilerParams(use_tc_tiling_on_sc=False),
        scratch_types=[
            pltpu.VMEM((IRING, CHUNK), jnp.int32),
            pltpu.VMEM((IRING, CHUNK), jnp.int32),
            [pltpu.VMEM((CHUNK, d), jnp.float32) for _ in range(RB)],
            pltpu.VMEM_SHARED((NACC, d), jnp.float32),
            [pltpu.SemaphoreType.DMA for _ in range(RB)],
            [pltpu.SemaphoreType.DMA for _ in range(RB)],
            [pltpu.SemaphoreType.DMA for _ in range(2)],
        ] + ([pltpu.VMEM_SHARED((N, d), jnp.float32),
              pltpu.VMEM_SHARED((N, d), jnp.float32)] if staged else []),
    )
    def spmm(tab0, tab1, srcs, dsts, out0, out1, sidx, didx, rows, acc,
             gsem, ssem, isem, *stabs):
        c = lax.axis_index("c")
        s = lax.axis_index("s")

        if staged:
            base = 0
            for sz in _chunks(OPT, BCH):
                row = s * OPT + base
                for th, ts in ((tab0, stabs[0]), (tab1, stabs[1])):
                    pltpu.sync_copy(th.at[pl.ds(row, sz)], rows[0].at[pl.ds(0, sz)])
                    pltpu.sync_copy(rows[0].at[pl.ds(0, sz)], ts.at[pl.ds(row, sz)])
                base += sz
            tab0, tab1 = stabs[0], stabs[1]

        def zero_row(i, _):
            for j in range(d // 16):
                rows[0][i, pl.ds(j * 16, 16)] = jnp.zeros((16,), jnp.float32)
            return 0

        lax.fori_loop(0, CHUNK, zero_row, 0)
        base = 0
        for sz in _chunks(ZPT, CHUNK):
            pltpu.sync_copy(rows[0].at[pl.ds(0, sz)],
                            acc.at[pl.ds(s * ZPT + base, sz)])
            base += sz
        plsc.subcore_barrier()

        for k in range(2):
            tab = tab0 if k == 0 else tab1
            pltpu.sync_copy(srcs.at[c, k, s, pl.ds(0, IRING)], sidx)
            pltpu.sync_copy(dsts.at[c, k, s, pl.ds(0, IRING)], didx)
            pltpu.make_async_copy(tab.at[sidx.at[0]], rows[0], gsem[0]).start()
            pltpu.make_async_copy(tab.at[sidx.at[1]], rows[1], gsem[1]).start()

            def body(i, _):
                j0 = RB * i
                for b in range(RB):
                    j = j0 + b
                    bn = (b + 2) % RB
                    @pl.when(j >= 2)
                    def _():
                        pltpu.make_async_copy(
                            rows[bn], acc.at[didx.at[0]], ssem[bn]).wait()

                    @pl.when(jnp.logical_and(j + 4 < CPT, (j + 4) % IB == 0))
                    def _():
                        blk = (j + 4) // IB
                        slot = lax.rem(blk, 2) * IB
                        row0 = blk * IB
                        pltpu.async_copy(srcs.at[c, k, s, pl.ds(row0, IB)],
                                         sidx.at[pl.ds(slot, IB)], isem[0])
                        pltpu.async_copy(dsts.at[c, k, s, pl.ds(row0, IB)],
                                         didx.at[pl.ds(slot, IB)], isem[1])

                    @pl.when(jnp.logical_and(j + 2 < CPT, (j + 2) % IB == 0))
                    def _():
                        pltpu.make_async_copy(
                            srcs.at[c, k, s, pl.ds(0, IB)],
                            sidx.at[pl.ds(0, IB)], isem[0]).wait()
                        pltpu.make_async_copy(
                            dsts.at[c, k, s, pl.ds(0, IB)],
                            didx.at[pl.ds(0, IB)], isem[1]).wait()

                    @pl.when(j + 2 < CPT)
                    def _():
                        r = lax.rem(j + 2, IRING)
                        pltpu.make_async_copy(
                            tab.at[sidx.at[r]], rows[bn], gsem[bn]).start()

                    pltpu.make_async_copy(tab.at[sidx.at[0]], rows[b],
                                          gsem[b]).wait()
                    pltpu.async_copy(rows[b], acc.at[didx.at[lax.rem(j, IRING)]],
                                     ssem[b], add=True)
                return 0

            lax.fori_loop(0, CPT // RB, body, 0)
            for j in (CPT - 2, CPT - 1):
                b = j % RB
                pltpu.make_async_copy(rows[b], acc.at[didx.at[0]], ssem[b]).wait()

        plsc.subcore_barrier()
        for cc, o in ((0, out0), (1, out1)):
            @pl.when(c == cc)
            def _():
                base = 0
                for sz in _chunks(OPT, BCH):
                    row = s * OPT + base
                    pltpu.sync_copy(acc.at[pl.ds(row, sz)], rows[0].at[pl.ds(0, sz)])
                    pltpu.sync_copy(rows[0].at[pl.ds(0, sz)], o.at[pl.ds(row, sz)])
                    base += sz

    return spmm


_spmm128 = _make_spmm(NHID)
_spmm32 = _make_spmm(D2P)


def kernel(x_0, x_1, ei_00, ei_01, ei_10, ei_11, W1_0, W1_1, bias1, W2, b2):
    f32 = jnp.float32
    npad = EPAD - E

    pad_src = jnp.arange(npad, dtype=jnp.int32) % N
    pad_dst = N + jnp.arange(npad, dtype=jnp.int32) % (NACC - N)

    def prep(ei):
        src = jnp.concatenate([ei[0], pad_src])
        dst = jnp.concatenate([ei[1], pad_dst])
        return src, dst

    s00, d00 = prep(ei_00)
    s01, d01 = prep(ei_01)
    s10, d10 = prep(ei_10)
    s11, d11 = prep(ei_11)
    srcs = jnp.stack([s00, s01, s10, s11]).reshape(2, 2, NTILES, CPT, CHUNK)
    dsts = jnp.stack([d00, d01, d10, d11]).reshape(2, 2, NTILES, CPT, CHUNK)

    br = 1000
    nb = N // br
    row_spec = pl.BlockSpec((br, D_IN), lambda i: (i, 0))
    w_spec = pl.BlockSpec((D_IN, NHID), lambda i: (0, 0))
    s0, s1 = pl.pallas_call(
        _mm_body,
        grid=(nb,),
        in_specs=[row_spec, row_spec, w_spec, w_spec],
        out_specs=[row_spec, row_spec],
        out_shape=[jax.ShapeDtypeStruct((N, NHID), f32)] * 2,
    )(x_0, x_1, W1_0, W1_1)

    y0, y1 = _spmm128(s0, s1, srcs, dsts)

    W2p = jnp.zeros((NHID, D2P), f32).at[:, :DIM2].set(W2)
    b2p = jnp.zeros((1, D2P), f32).at[0, :DIM2].set(b2)
    o_spec = pl.BlockSpec((br, D2P), lambda i: (i, 0))
    t0, t1 = pl.pallas_call(
        _layer2_body,
        grid=(nb,),
        in_specs=[
            row_spec, row_spec,
            pl.BlockSpec((1, NHID), lambda i: (0, 0)),
            pl.BlockSpec((NHID, D2P), lambda i: (0, 0)),
            pl.BlockSpec((1, D2P), lambda i: (0, 0)),
        ],
        out_specs=[o_spec, o_spec],
        out_shape=[jax.ShapeDtypeStruct((N, D2P), f32)] * 2,
    )(y0, y1, bias1.reshape(1, NHID), W2p, b2p)

    z0, z1 = _spmm32(t0, t1, srcs, dsts)

    return tuple(pl.pallas_call(
        _logsoftmax_body,
        grid=(nb,),
        in_specs=[o_spec, o_spec],
        out_specs=[pl.BlockSpec((br, DIM2), lambda i: (i, 0))] * 2,
        out_shape=[jax.ShapeDtypeStruct((N, DIM2), f32)] * 2,
    )(z0, z1))


# speedup vs baseline: 1.0544x; 1.0544x over previous
import functools

import jax
import jax.numpy as jnp
from jax import lax
from jax.experimental import pallas as pl
from jax.experimental.pallas import tpu as pltpu
from jax.experimental.pallas import tpu_sc as plsc

N = 10000
E = 320000
D_IN = 128
NHID = 128
NCLASS = 16
DIM2 = NCLASS + 2
D2P = 32

NTILES = 16
CHUNK = 64
RB = 4
CPT = 320
IB = 20
NIB = CPT // IB
IRING = 2 * IB
EPT = CPT * CHUNK
EPAD = EPT * NTILES
NACC = 10016
ZPT = NACC // NTILES
OPT = N // NTILES
BCH = CHUNK


def _mm_body(x0_ref, x1_ref, w0_ref, w1_ref, o0_ref, o1_ref):
    o0_ref[...] = jnp.dot(x0_ref[...], w0_ref[...], preferred_element_type=jnp.float32)
    o1_ref[...] = jnp.dot(x1_ref[...], w1_ref[...], preferred_element_type=jnp.float32)


def _layer2_body(y0_ref, y1_ref, b1_ref, w_ref, b2_ref, o0_ref, o1_ref):
    for y_ref, o_ref in ((y0_ref, o0_ref), (y1_ref, o1_ref)):
        h = jnp.maximum(y_ref[...] + 2.0 * b1_ref[...], 0.0)
        o_ref[...] = jnp.dot(h, w_ref[...], preferred_element_type=jnp.float32) + b2_ref[...]


def _logsoftmax_body(z0_ref, z1_ref, o0_ref, o1_ref):
    for z_ref, o_ref in ((z0_ref, o0_ref), (z1_ref, o1_ref)):
        z = z_ref[...]
        col = lax.broadcasted_iota(jnp.int32, z.shape, 1)
        valid = col < DIM2
        zm = jnp.where(valid, z, -jnp.inf)
        m = jnp.max(zm, axis=1, keepdims=True)
        e = jnp.where(valid, jnp.exp(z - m), 0.0)
        s = jnp.sum(e, axis=1, keepdims=True)
        o_ref[...] = ((z - m) - jnp.log(s))[:, :DIM2]


def _chunks(total, step):
    out = []
    while total > 0:
        out.append(min(step, total))
        total -= out[-1]
    return tuple(out)


def _make_spmm(d, staged=False):
    mesh = plsc.VectorSubcoreMesh(core_axis_name="c", subcore_axis_name="s")

    @functools.partial(
        pl.kernel,
        out_type=[jax.ShapeDtypeStruct((N, d), jnp.float32),
                  jax.ShapeDtypeStruct((N, d), jnp.float32)],
        mesh=mesh,
        compiler_params=pltpu.CompilerParams(use_tc_tiling_on_sc=False),
        scratch_types=[
            pltpu.VMEM((IRING, CHUNK), jnp.int32),
            pltpu.VMEM((IRING, CHUNK), jnp.int32),
            [pltpu.VMEM((CHUNK, d), jnp.float32) for _ in range(RB)],
            pltpu.VMEM_SHARED((NACC, d), jnp.float32),
            [pltpu.SemaphoreType.DMA for _ in range(RB)],
            [pltpu.SemaphoreType.DMA for _ in range(RB)],
            [pltpu.SemaphoreType.DMA for _ in range(2)],
        ] + ([pltpu.VMEM_SHARED((N, d), jnp.float32),
              pltpu.VMEM_SHARED((N, d), jnp.float32)] if staged else []),
    )
    def spmm(tab0, tab1, srcs, dsts, out0, out1, sidx, didx, rows, acc,
             gsem, ssem, isem, *stabs):
        c = lax.axis_index("c")
        s = lax.axis_index("s")

        if staged:
            base = 0
            for sz in _chunks(OPT, BCH):
                row = s * OPT + base
                for th, ts in ((tab0, stabs[0]), (tab1, stabs[1])):
                    pltpu.sync_copy(th.at[pl.ds(row, sz)], rows[0].at[pl.ds(0, sz)])
                    pltpu.sync_copy(rows[0].at[pl.ds(0, sz)], ts.at[pl.ds(row, sz)])
                base += sz
            tab0, tab1 = stabs[0], stabs[1]

        def zero_row(i, _):
            for j in range(d // 16):
                rows[0][i, pl.ds(j * 16, 16)] = jnp.zeros((16,), jnp.float32)
            return 0

        lax.fori_loop(0, CHUNK, zero_row, 0)
        base = 0
        for sz in _chunks(ZPT, CHUNK):
            pltpu.sync_copy(rows[0].at[pl.ds(0, sz)],
                            acc.at[pl.ds(s * ZPT + base, sz)])
            base += sz
        plsc.subcore_barrier()

        for k in range(2):
            tab = tab0 if k == 0 else tab1
            pltpu.sync_copy(srcs.at[c, k, s, pl.ds(0, IRING)], sidx)
            pltpu.sync_copy(dsts.at[c, k, s, pl.ds(0, IRING)], didx)
            pltpu.make_async_copy(tab.at[sidx.at[0]], rows[0], gsem[0]).start()
            pltpu.make_async_copy(tab.at[sidx.at[1]], rows[1], gsem[1]).start()

            def body(i, _):
                j0 = RB * i
                for b in range(RB):
                    j = j0 + b
                    bn = (b + 2) % RB
                    @pl.when(j >= 2)
                    def _():
                        pltpu.make_async_copy(
                            rows[bn], acc.at[didx.at[0]], ssem[bn]).wait()

                    @pl.when(jnp.logical_and(j + 4 < CPT, (j + 4) % IB == 0))
                    def _():
                        blk = (j + 4) // IB
                        slot = lax.rem(blk, 2) * IB
                        row0 = blk * IB
                        pltpu.async_copy(srcs.at[c, k, s, pl.ds(row0, IB)],
                                         sidx.at[pl.ds(slot, IB)], isem[0])
                        pltpu.async_copy(dsts.at[c, k, s, pl.ds(row0, IB)],
                                         didx.at[pl.ds(slot, IB)], isem[1])

                    @pl.when(jnp.logical_and(j + 2 < CPT, (j + 2) % IB == 0))
                    def _():
                        pltpu.make_async_copy(
                            srcs.at[c, k, s, pl.ds(0, IB)],
                            sidx.at[pl.ds(0, IB)], isem[0]).wait()
                        pltpu.make_async_copy(
                            dsts.at[c, k, s, pl.ds(0, IB)],
                            didx.at[pl.ds(0, IB)], isem[1]).wait()

                    @pl.when(j + 2 < CPT)
                    def _():
                        r = lax.rem(j + 2, IRING)
                        pltpu.make_async_copy(
                            tab.at[sidx.at[r]], rows[bn], gsem[bn]).start()

                    pltpu.make_async_copy(tab.at[sidx.at[0]], rows[b],
                                          gsem[b]).wait()
                    pltpu.async_copy(rows[b], acc.at[didx.at[lax.rem(j, IRING)]],
                                     ssem[b], add=True)
                return 0

            lax.fori_loop(0, CPT // RB, body, 0)
            for j in (CPT - 2, CPT - 1):
                b = j % RB
                pltpu.make_async_copy(rows[b], acc.at[didx.at[0]], ssem[b]).wait()

        plsc.subcore_barrier()
        for cc, o in ((0, out0), (1, out1)):
            @pl.when(c == cc)
            def _():
                base = 0
                for sz in _chunks(OPT, BCH):
                    row = s * OPT + base
                    pltpu.sync_copy(acc.at[pl.ds(row, sz)], rows[0].at[pl.ds(0, sz)])
                    pltpu.sync_copy(rows[0].at[pl.ds(0, sz)], o.at[pl.ds(row, sz)])
                    base += sz

    return spmm


_spmm128 = _make_spmm(NHID)
_spmm32 = _make_spmm(D2P, staged=True)


def kernel(x_0, x_1, ei_00, ei_01, ei_10, ei_11, W1_0, W1_1, bias1, W2, b2):
    f32 = jnp.float32
    npad = EPAD - E

    pad_src = jnp.arange(npad, dtype=jnp.int32) % N
    pad_dst = N + jnp.arange(npad, dtype=jnp.int32) % (NACC - N)

    def prep(ei):
        src = jnp.concatenate([ei[0], pad_src])
        dst = jnp.concatenate([ei[1], pad_dst])
        return src, dst

    s00, d00 = prep(ei_00)
    s01, d01 = prep(ei_01)
    s10, d10 = prep(ei_10)
    s11, d11 = prep(ei_11)
    srcs = jnp.stack([s00, s01, s10, s11]).reshape(2, 2, NTILES, CPT, CHUNK)
    dsts = jnp.stack([d00, d01, d10, d11]).reshape(2, 2, NTILES, CPT, CHUNK)

    br = 1000
    nb = N // br
    row_spec = pl.BlockSpec((br, D_IN), lambda i: (i, 0))
    w_spec = pl.BlockSpec((D_IN, NHID), lambda i: (0, 0))
    s0, s1 = pl.pallas_call(
        _mm_body,
        grid=(nb,),
        in_specs=[row_spec, row_spec, w_spec, w_spec],
        out_specs=[row_spec, row_spec],
        out_shape=[jax.ShapeDtypeStruct((N, NHID), f32)] * 2,
    )(x_0, x_1, W1_0, W1_1)

    y0, y1 = _spmm128(s0, s1, srcs, dsts)

    W2p = jnp.zeros((NHID, D2P), f32).at[:, :DIM2].set(W2)
    b2p = jnp.zeros((1, D2P), f32).at[0, :DIM2].set(b2)
    o_spec = pl.BlockSpec((br, D2P), lambda i: (i, 0))
    t0, t1 = pl.pallas_call(
        _layer2_body,
        grid=(nb,),
        in_specs=[
            row_spec, row_spec,
            pl.BlockSpec((1, NHID), lambda i: (0, 0)),
            pl.BlockSpec((NHID, D2P), lambda i: (0, 0)),
            pl.BlockSpec((1, D2P), lambda i: (0, 0)),
        ],
        out_specs=[o_spec, o_spec],
        out_shape=[jax.ShapeDtypeStruct((N, D2P), f32)] * 2,
    )(y0, y1, bias1.reshape(1, NHID), W2p, b2p)

    z0, z1 = _spmm32(t0, t1, srcs, dsts)

    return tuple(pl.pallas_call(
        _logsoftmax_body,
        grid=(nb,),
        in_specs=[o_spec, o_spec],
        out_specs=[pl.BlockSpec((br, DIM2), lambda i: (i, 0))] * 2,
        out_shape=[jax.ShapeDtypeStruct((N, DIM2), f32)] * 2,
    )(z0, z1))
